# baseline (speedup 1.0000x reference)
import functools

import jax
import jax.numpy as jnp
from jax import lax
from jax.experimental import pallas as pl
from jax.experimental.pallas import tpu as pltpu
from jax.experimental.pallas import tpu_sc as plsc

VOCAB = 1000000
D = 64
H = 50
O = 30
B = 4096
L = 50
DELTA = 1.0

NC = 2
NS = 16
NW = NC * NS

POOL_ARRAYS = 7
MLP_ARRAYS = 3
ARRAYS = POOL_ARRAYS + MLP_ARRAYS
BCHUNK = B // NW
NPAIR = BCHUNK // 2

POOL_ROWS = POOL_ARRAYS * B // 2
GATH_ROWS = MLP_ARRAYS * (B // 2) * L


@functools.cache
def _sc_gather():
  mesh = plsc.VectorSubcoreMesh(core_axis_name="c", subcore_axis_name="s")
  return pl.kernel(
      _sc_body,
      mesh=mesh,
      compiler_params=pltpu.CompilerParams(use_tc_tiling_on_sc=False),
      out_type=[
          jax.ShapeDtypeStruct((POOL_ROWS, 2 * D), jnp.float32),
          jax.ShapeDtypeStruct((GATH_ROWS, 2 * D), jnp.float32),
      ],
      scratch_types=[
          pltpu.VMEM((BCHUNK, L), jnp.int32),
          pltpu.VMEM((L, D), jnp.float32),
          pltpu.VMEM((L, D), jnp.float32),
          pltpu.VMEM((NPAIR, 2 * D), jnp.float32),
          pltpu.SemaphoreType.DMA,
          pltpu.SemaphoreType.DMA,
      ],
  )


def _sc_body(emb, idx_all, pool_out, gath_out, idx_s, buf_a, buf_b, outc,
             sem_a, sem_b):
  wid = lax.axis_index("s") * NC + lax.axis_index("c")

  def fire(idx_row, buf, sem):
    pltpu.make_async_copy(emb.at[idx_row], buf, sem).start()

  def wait(idx_row, buf, sem):
    pltpu.make_async_copy(emb.at[idx_row], buf, sem).wait()

  def accum50(buf):
    def body(i, accs):
      return tuple(accs[j] + buf[i, pl.ds(16 * j, 16)] for j in range(4))
    z = jnp.zeros((16,), jnp.float32)
    return lax.fori_loop(0, L, body, (z, z, z, z))

  def run_half(t0, nt, process):
    fire(idx_s.at[t0], buf_a, sem_a)

    def pair(g, _):
      ta = t0 + 2 * g
      tb = ta + 1
      fire(idx_s.at[tb], buf_b, sem_b)
      wait(idx_s.at[ta], buf_a, sem_a)
      process(buf_a, ta)

      @pl.when(2 * g + 2 < nt)
      def _():
        fire(idx_s.at[ta + 2], buf_a, sem_a)

      wait(idx_s.at[tb], buf_b, sem_b)
      process(buf_b, tb)
      return 0

    lax.fori_loop(0, nt // 2, pair, 0)

  def stage_load(arr):
    pltpu.sync_copy(idx_all.at[pl.ds(B * arr + BCHUNK * wid, BCHUNK)], idx_s)

  for arr in range(POOL_ARRAYS):
    stage_load(arr)

    def pool_proc_lo(buf, t):
      a = accum50(buf)
      for j in range(4):
        outc[t, pl.ds(16 * j, 16)] = a[j]

    def pool_proc_hi(buf, t):
      a = accum50(buf)
      for j in range(4):
        outc[t - NPAIR, pl.ds(D + 16 * j, 16)] = a[j]

    run_half(0, NPAIR, pool_proc_lo)
    run_half(NPAIR, NPAIR, pool_proc_hi)
    pltpu.sync_copy(
        outc, pool_out.at[pl.ds((B // 2) * arr + NPAIR * wid, NPAIR)])

  for m in range(MLP_ARRAYS):
    stage_load(POOL_ARRAYS + m)
    prow0 = L * NPAIR * (NW * m + wid)

    def mlp_proc_lo(buf, t):
      pltpu.sync_copy(
          buf, gath_out.at[pl.ds(prow0 + L * t, L), pl.ds(0, D)])

    def mlp_proc_hi(buf, t):
      pltpu.sync_copy(
          buf, gath_out.at[pl.ds(prow0 + L * (t - NPAIR), L), pl.ds(D, D)])

    run_half(0, NPAIR, mlp_proc_lo)
    run_half(NPAIR, NPAIR, mlp_proc_hi)


def _tc_body(pooled_ref, gath_ref, whw_ref, whb_ref, wpw_ref, wpb_ref,
             out_ref):
  inv = 1.0 / (L * L)
  whw = whw_ref[...]
  whb = whb_ref[...]
  wpw = wpw_ref[...]
  wpb = wpb_ref[...]

  def proj(e):
    z = lax.dot_general(e, whw, (((1,), (1,)), ((), ())),
                        preferred_element_type=jnp.float32) + whb
    h = 1.0 / (1.0 + jnp.exp(-z))
    return lax.dot_general(h, wpw, (((1,), (1,)), ((), ())),
                           preferred_element_type=jnp.float32) + wpb

  qv = pooled_ref[0]
  for half in range(2):
    lo, hi = D * half, D * half + D
    qv_h = qv[:, lo:hi]
    tot = jnp.zeros((NPAIR,), jnp.float32)
    for k in range(3):
      dq = jnp.sum(qv_h * pooled_ref[1 + 2 * k][:, lo:hi], axis=1)
      dn = jnp.sum(qv_h * pooled_ref[2 + 2 * k][:, lo:hi], axis=1)
      tot = tot + jnp.maximum(DELTA - inv * dq + inv * dn, 0.0)

    def rowdots(x, y):
      p = (x * y).reshape(NPAIR, L, O)
      return jnp.sum(jnp.sum(p, axis=2), axis=1)

    oq = proj(gath_ref[0, 0][:, :, lo:hi].reshape(NPAIR * L, D))
    op_ = proj(gath_ref[1, 0][:, :, lo:hi].reshape(NPAIR * L, D))
    on_ = proj(gath_ref[2, 0][:, :, lo:hi].reshape(NPAIR * L, D))
    dqp = rowdots(oq, op_)
    dqn = rowdots(oq, on_)
    out_ref[0, half, :] = tot + jnp.maximum(DELTA - dqp + dqn, 0.0)


def _tc_call(pooled3, gath5, wh_w, wh_b2, wp_w, wp_b2):
  return pl.pallas_call(
      _tc_body,
      grid=(NW,),
      in_specs=[
          pl.BlockSpec((POOL_ARRAYS, NPAIR, 2 * D), lambda i: (0, i, 0)),
          pl.BlockSpec((MLP_ARRAYS, 1, NPAIR, L, 2 * D),
                       lambda i: (0, i, 0, 0, 0)),
          pl.BlockSpec((H, D), lambda i: (0, 0)),
          pl.BlockSpec((1, H), lambda i: (0, 0)),
          pl.BlockSpec((O, H), lambda i: (0, 0)),
          pl.BlockSpec((1, O), lambda i: (0, 0)),
      ],
      out_specs=pl.BlockSpec((1, 2, NPAIR), lambda i: (i, 0, 0)),
      out_shape=jax.ShapeDtypeStruct((NW, 2, NPAIR), jnp.float32),
  )(pooled3, gath5, wh_w, wh_b2, wp_w, wp_b2)


def kernel(q_v, q_a0, n_a0, q_a1, n_a1, q_a2, n_a2, query, pos, neg,
           emb, wh_w, wh_b, wp_w, wp_b):
  idx_all = jnp.concatenate(
      [q_v, q_a0, n_a0, q_a1, n_a1, q_a2, n_a2, query, pos, neg],
      axis=0).astype(jnp.int32)

  pooled, gath = _sc_gather()(emb, idx_all)
  pooled3 = pooled.reshape(POOL_ARRAYS, B // 2, 2 * D)
  gath5 = gath.reshape(MLP_ARRAYS, NW, NPAIR, L, 2 * D)

  out = _tc_call(pooled3, gath5, wh_w, wh_b.reshape(1, H),
                 wp_w, wp_b.reshape(1, O))
  return out.reshape(B)
